# baseline (speedup 1.0000x reference)
import functools

import jax
import jax.numpy as jnp
from jax.experimental import pallas as pl
from jax.experimental.pallas import tpu as pltpu

D_MODEL_ = 2048
N_EXP_ = 64
BM_ = 1024
NBUF_ = 4
NSUB_ = 4
SUB_ = BM_ // NSUB_


def _router_body(x_hbm, wt_ref, b_ref, o_ref, xbuf, sem, *, n_steps):
    def start(i):
        slot = i % NBUF_
        for j in range(NSUB_):
            pltpu.make_async_copy(
                x_hbm.at[pl.ds(i * BM_ + j * SUB_, SUB_), :],
                xbuf.at[slot, pl.ds(j * SUB_, SUB_)],
                sem.at[slot, j],
            ).start(priority=j % 2)

    def wait(i):
        slot = i % NBUF_
        for j in range(NSUB_):
            pltpu.make_async_copy(
                x_hbm.at[pl.ds(i * BM_ + j * SUB_, SUB_), :],
                xbuf.at[slot, pl.ds(j * SUB_, SUB_)],
                sem.at[slot, j],
            ).wait()

    for i in range(min(NBUF_, n_steps)):
        start(i)
    bias = b_ref[...]
    for i in range(n_steps):
        wait(i)
        o_ref[pl.ds(i * BM_, BM_), :] = (
            jnp.dot(xbuf[i % NBUF_], wt_ref[...], preferred_element_type=jnp.float32)
            + bias
        )
        if i + NBUF_ < n_steps:
            start(i + NBUF_)


def kernel(x, W, b):
    bsz, seq, d = x.shape
    m = bsz * seq
    n_steps = m // BM_
    x2 = x.reshape(m, d)
    wt = W.T
    b2 = b.reshape(1, N_EXP_)
    out = pl.pallas_call(
        functools.partial(_router_body, n_steps=n_steps),
        in_specs=[
            pl.BlockSpec(memory_space=pltpu.MemorySpace.HBM),
            pl.BlockSpec(memory_space=pltpu.VMEM),
            pl.BlockSpec(memory_space=pltpu.VMEM),
        ],
        out_specs=pl.BlockSpec(memory_space=pltpu.VMEM),
        out_shape=jax.ShapeDtypeStruct((m, N_EXP_), jnp.float32),
        scratch_shapes=[
            pltpu.VMEM((NBUF_, BM_, d), jnp.float32),
            pltpu.SemaphoreType.DMA((NBUF_, NSUB_)),
        ],
    )(x2, wt, b2)
    return out.reshape(bsz, seq, N_EXP_)


# speedup vs baseline: 1.1666x; 1.1666x over previous
import jax
import jax.numpy as jnp
from jax.experimental import pallas as pl
from jax.experimental.pallas import tpu as pltpu

D_MODEL_ = 2048
N_EXP_ = 64
BM_ = 1024


def _router_body(x_ref, w_ref, b_ref, o_ref):
    acc = jax.lax.dot_general(
        x_ref[...],
        w_ref[...],
        (((1,), (1,)), ((), ())),
        preferred_element_type=jnp.float32,
    )
    o_ref[...] = acc + b_ref[...].reshape(1, N_EXP_)


def kernel(x, W, b):
    bsz, seq, d = x.shape
    m = bsz * seq
    x2 = x.reshape(m, d)
    grid = (m // BM_,)
    out = pl.pallas_call(
        _router_body,
        grid=grid,
        in_specs=[
            pl.BlockSpec((BM_, d), lambda i: (i, 0)),
            pl.BlockSpec((N_EXP_, d), lambda i: (0, 0)),
            pl.BlockSpec((N_EXP_,), lambda i: (0,)),
        ],
        out_specs=pl.BlockSpec((BM_, N_EXP_), lambda i: (i, 0)),
        out_shape=jax.ShapeDtypeStruct((m, N_EXP_), jnp.float32),
        compiler_params=pltpu.CompilerParams(
            dimension_semantics=("arbitrary",),
        ),
    )(x2, W, b)
    return out.reshape(bsz, seq, N_EXP_)
